# baseline (speedup 1.0000x reference)
import functools

import jax
import jax.numpy as jnp
from jax import lax
from jax.experimental import pallas as pl
from jax.experimental.pallas import tpu as pltpu
from jax.experimental.pallas import tpu_sc as plsc

B = 4096
D = 768
K = 8192
PLEN = 20
HALF = (PLEN // 2) * D

BM = 2048
BK = 1024
NB = B // BM
NK = K // BK
EPS = 1e-12


def _topk_body(xq_ref, key_ref, out_ref, best_ref, bidx_ref):
    kb = pl.program_id(1)
    xq = xq_ref[...]
    qn = xq / jnp.maximum(jnp.sqrt(jnp.sum(xq * xq, axis=1, keepdims=True)), EPS)
    kv = key_ref[...]
    kn = kv / jnp.maximum(jnp.sqrt(jnp.sum(kv * kv, axis=1, keepdims=True)), EPS)
    s = lax.dot_general(qn, kn, (((1,), (1,)), ((), ())),
                        preferred_element_type=jnp.float32)
    m = jnp.max(s, axis=1, keepdims=True)
    iota = lax.broadcasted_iota(jnp.int32, s.shape, 1)
    a = jnp.min(jnp.where(s == m, iota, K), axis=1, keepdims=True) + kb * BK

    @pl.when(kb == 0)
    def _():
        best_ref[...] = m
        bidx_ref[...] = a

    @pl.when(kb > 0)
    def _():
        prev_best = best_ref[...]
        prev_idx = bidx_ref[...]
        upd = m > prev_best
        best_ref[...] = jnp.where(upd, m, prev_best)
        bidx_ref[...] = jnp.where(upd, a, prev_idx)

    @pl.when(kb == NK - 1)
    def _():
        out_ref[...] = bidx_ref[...]


def _top1_indices(xq, key):
    return pl.pallas_call(
        _topk_body,
        grid=(NB, NK),
        in_specs=[
            pl.BlockSpec((BM, D), lambda b, k: (b, 0)),
            pl.BlockSpec((BK, D), lambda b, k: (k, 0)),
        ],
        out_specs=pl.BlockSpec((BM, 1), lambda b, k: (b, 0)),
        out_shape=jax.ShapeDtypeStruct((B, 1), jnp.int32),
        scratch_shapes=[
            pltpu.VMEM((BM, 1), jnp.float32),
            pltpu.VMEM((BM, 1), jnp.int32),
        ],
    )(xq, key)


NC = 2
NS = 16
NW = NC * NS
BPW = B // NW
CH1 = 4
NCH1 = BPW // CH1
CH2 = 16
NCH2 = BPW // CH2


@functools.cache
def _make_sc_both():
    @functools.partial(
        pl.kernel,
        out_type=(
            jax.ShapeDtypeStruct((B, 16, D), jnp.float32),
            jax.ShapeDtypeStruct((B, 8, D), jnp.float32),
        ),
        mesh=plsc.VectorSubcoreMesh(core_axis_name="c", subcore_axis_name="s"),
        scratch_types=[
            pltpu.VMEM((NCH1, CH1), jnp.int32),
            pltpu.VMEM((CH1, 16, D), jnp.float32),
            pltpu.VMEM((CH1, 8, D), jnp.float32),
            pltpu.SemaphoreType.DMA,
            pltpu.SemaphoreType.DMA,
        ],
    )
    def _sc_both(p0_hbm, idx_hbm, pkx_hbm, pvc_hbm, idx_v, bufa, bufb,
                 sema, semb):
        wid = lax.axis_index("s") * NC + lax.axis_index("c")
        base = wid * BPW
        pltpu.sync_copy(idx_hbm.at[wid], idx_v)

        def body(j, carry):
            ids = idx_v.at[j]
            a = pltpu.async_copy(p0_hbm.at[ids, pl.ds(0, 16), :], bufa, sema)
            b = pltpu.async_copy(p0_hbm.at[ids, pl.ds(16, 8), :], bufb, semb)
            rows = pl.ds(base + j * CH1, CH1)
            a.wait()
            pltpu.sync_copy(bufa, pkx_hbm.at[rows])
            b.wait()
            pltpu.sync_copy(bufb, pvc_hbm.at[rows])
            return carry

        lax.fori_loop(0, NCH1, body, 0)

    return _sc_both


def kernel(l, x_block, x_query, key, p0):
    del l
    xq = x_query.reshape(B, D)
    idx = _top1_indices(xq, key)
    p0p = jnp.concatenate([p0, p0[:, :4, :]], axis=1)
    pkx, pvc = _make_sc_both()(p0p, idx.reshape(NW, NCH1, CH1))
    pk = pkx[:, :PLEN // 2, :]
    pv = jnp.concatenate([pkx[:, PLEN // 2:16, :], pvc[:, :4, :]], axis=1)
    return (pk, pv, x_block)


# speedup vs baseline: 1.1856x; 1.1856x over previous
import functools

import jax
import jax.numpy as jnp
from jax import lax
from jax.experimental import pallas as pl
from jax.experimental.pallas import tpu as pltpu
from jax.experimental.pallas import tpu_sc as plsc

B = 4096
D = 768
K = 8192
PLEN = 20
HALF = (PLEN // 2) * D

BM = 2048
BK = 1024
NB = B // BM
NK = K // BK
EPS = 1e-12


def _topk_body(xq_ref, key_ref, out_ref, best_ref, bidx_ref):
    kb = pl.program_id(1)
    xq = xq_ref[...]
    qn = xq / jnp.maximum(jnp.sqrt(jnp.sum(xq * xq, axis=1, keepdims=True)), EPS)
    kv = key_ref[...]
    kn = kv / jnp.maximum(jnp.sqrt(jnp.sum(kv * kv, axis=1, keepdims=True)), EPS)
    s = lax.dot_general(qn, kn, (((1,), (1,)), ((), ())),
                        preferred_element_type=jnp.float32)
    m = jnp.max(s, axis=1, keepdims=True)
    iota = lax.broadcasted_iota(jnp.int32, s.shape, 1)
    a = jnp.min(jnp.where(s == m, iota, K), axis=1, keepdims=True) + kb * BK

    @pl.when(kb == 0)
    def _():
        best_ref[...] = m
        bidx_ref[...] = a

    @pl.when(kb > 0)
    def _():
        prev_best = best_ref[...]
        prev_idx = bidx_ref[...]
        upd = m > prev_best
        best_ref[...] = jnp.where(upd, m, prev_best)
        bidx_ref[...] = jnp.where(upd, a, prev_idx)

    @pl.when(kb == NK - 1)
    def _():
        out_ref[...] = bidx_ref[...]


def _top1_indices(xq, key):
    return pl.pallas_call(
        _topk_body,
        grid=(NB, NK),
        in_specs=[
            pl.BlockSpec((BM, D), lambda b, k: (b, 0)),
            pl.BlockSpec((BK, D), lambda b, k: (k, 0)),
        ],
        out_specs=pl.BlockSpec((BM, 1), lambda b, k: (b, 0)),
        out_shape=jax.ShapeDtypeStruct((B, 1), jnp.int32),
        scratch_shapes=[
            pltpu.VMEM((BM, 1), jnp.float32),
            pltpu.VMEM((BM, 1), jnp.int32),
        ],
    )(xq, key)


NC = 2
NS = 16
NW = NC * NS
BPW = B // NW
CH1 = 4
NCH1 = BPW // CH1
CH2 = 16
NCH2 = BPW // CH2


@functools.cache
def _make_sc_main():
    @functools.partial(
        pl.kernel,
        out_type=jax.ShapeDtypeStruct((B, 16, D), jnp.float32),
        mesh=plsc.VectorSubcoreMesh(core_axis_name="c", subcore_axis_name="s"),
        scratch_types=[
            pltpu.VMEM((NCH1, CH1), jnp.int32),
            pltpu.VMEM((CH1, 16, D), jnp.float32),
            pltpu.VMEM((CH1, 16, D), jnp.float32),
            pltpu.SemaphoreType.DMA,
            pltpu.SemaphoreType.DMA,
        ],
    )
    def _sc_main(p0_hbm, idx_hbm, out_hbm, idx_v, buf0, buf1, sem0, sem1):
        wid = lax.axis_index("s") * NC + lax.axis_index("c")
        base = wid * BPW
        pltpu.sync_copy(idx_hbm.at[wid], idx_v)

        def body(jj, carry):
            j0 = jj * 2
            j1 = j0 + 1
            a = pltpu.async_copy(
                p0_hbm.at[idx_v.at[j0], pl.ds(0, 16), :], buf0, sem0)
            b = pltpu.async_copy(
                p0_hbm.at[idx_v.at[j1], pl.ds(0, 16), :], buf1, sem1)
            a.wait()
            pltpu.sync_copy(buf0, out_hbm.at[pl.ds(base + j0 * CH1, CH1)])
            b.wait()
            pltpu.sync_copy(buf1, out_hbm.at[pl.ds(base + j1 * CH1, CH1)])
            return carry

        lax.fori_loop(0, NCH1 // 2, body, 0)

    return _sc_main


@functools.cache
def _make_sc_tail():
    @functools.partial(
        pl.kernel,
        out_type=jax.ShapeDtypeStruct((B, 8, D), jnp.float32),
        mesh=plsc.VectorSubcoreMesh(core_axis_name="c", subcore_axis_name="s"),
        scratch_types=[
            pltpu.VMEM((NCH2, CH2), jnp.int32),
            pltpu.VMEM((CH2, 8, D), jnp.float32),
            pltpu.SemaphoreType.DMA,
        ],
    )
    def _sc_tail(r16_hbm, idx_hbm, out_hbm, idx_v, buf, sem):
        wid = lax.axis_index("s") * NC + lax.axis_index("c")
        base = wid * BPW
        pltpu.sync_copy(idx_hbm.at[wid], idx_v)

        def body(j, carry):
            ids = idx_v.at[j]
            pltpu.async_copy(r16_hbm.at[ids], buf, sem).wait()
            pltpu.sync_copy(buf, out_hbm.at[pl.ds(base + j * CH2, CH2)])
            return carry

        lax.fori_loop(0, NCH2, body, 0)

    return _sc_tail


def kernel(l, x_block, x_query, key, p0):
    del l
    xq = x_query.reshape(B, D)
    idx = _top1_indices(xq, key)
    r16 = jnp.pad(p0[:, 16:, :], ((0, 0), (0, 4), (0, 0)))
    pkx = _make_sc_main()(p0, idx.reshape(NW, NCH1, CH1))
    pvc = _make_sc_tail()(r16, idx.reshape(NW, NCH2, CH2))
    pk = pkx[:, :PLEN // 2, :]
    pv = jnp.concatenate([pkx[:, PLEN // 2:16, :], pvc[:, :4, :]], axis=1)
    return (pk, pv, x_block)
